# baseline (speedup 1.0000x reference)
import functools

import jax
import jax.numpy as jnp
from jax.experimental import pallas as pl
from jax.experimental.pallas import tpu as pltpu

_LN2 = 0.6931471805599453


def _comb(a1, b1, c1, d1, k1, a2, b2, c2, d2, k2):
    na = a2 * a1 + b2 * c1
    nb = a2 * b1 + b2 * d1
    nc = c2 * a1 + d2 * c1
    nd = c2 * b1 + d2 * d1
    m = jnp.maximum(jnp.maximum(na, nb), jnp.maximum(nc, nd))
    ebits = jax.lax.shift_right_logical(
        jax.lax.bitcast_convert_type(m, jnp.int32), 23)
    scale = jax.lax.bitcast_convert_type(
        jax.lax.shift_left(254 - ebits, 23), jnp.float32)
    nk = k1 + k2 + (ebits - 127)
    return na * scale, nb * scale, nc * scale, nd * scale, nk


def _body(lp_ref, len_ref, sm_ref, out_ref, acc_ref, kacc_ref,
          *, n_b, t_blk, n_seg, nv):
    i = pl.program_id(0)
    nsteps = pl.num_programs(0)
    lens = len_ref[...]
    seglen = t_blk // n_seg

    r_iota = jax.lax.broadcasted_iota(jnp.int32, (t_blk, n_b), 0)
    tglob = i * t_blk + (r_iota % n_seg) * seglen + r_iota // n_seg
    mask = tglob < lens

    labs = lp_ref[nv]

    s00 = jnp.zeros((t_blk, n_b), jnp.float32)
    s10 = jnp.zeros((t_blk, n_b), jnp.float32)
    emit = jnp.zeros((t_blk, n_b), jnp.float32)
    p_i = None
    for v in range(nv):
        ev = lp_ref[v]
        emit = jnp.where(labs == float(v), ev, emit)
        if v == 0:
            continue
        pv = jnp.exp(ev)
        if v == 1:
            s00 = s00 + sm_ref[3] * pv
        elif v == 2:
            p_i = pv
        else:
            s00 = s00 + sm_ref[4 + v - 3] * pv
            s10 = s10 + sm_ref[14 + v - 3] * pv

    num_part = jnp.sum(jnp.where(mask, emit, 0.0), axis=0, keepdims=True)

    a = jnp.where(mask, s00, 1.0)
    b = jnp.where(mask, s10, 0.0)
    c = jnp.where(mask, sm_ref[0] * p_i, 0.0)
    d = jnp.where(mask, sm_ref[1] * p_i, 1.0)

    a3 = a.reshape(seglen, n_seg, n_b)
    b3 = b.reshape(seglen, n_seg, n_b)
    c3 = c.reshape(seglen, n_seg, n_b)
    d3 = d.reshape(seglen, n_seg, n_b)

    ra, rb, rc, rd = a3[0], b3[0], c3[0], d3[0]
    rk = jnp.zeros((n_seg, n_b), jnp.int32)
    zk = rk
    for o in range(1, seglen):
        ra, rb, rc, rd, rk = _comb(ra, rb, rc, rd, rk,
                                   a3[o], b3[o], c3[o], d3[o], zk)

    pa, pb, pc, pd = ra[0:1], rb[0:1], rc[0:1], rd[0:1]
    pk = rk[0:1]
    for s in range(1, n_seg):
        pa, pb, pc, pd, pk = _comb(pa, pb, pc, pd, pk,
                                   ra[s:s + 1], rb[s:s + 1],
                                   rc[s:s + 1], rd[s:s + 1], rk[s:s + 1])

    @pl.when(i == 0)
    def _init():
        acc_ref[0:1] = pa
        acc_ref[1:2] = pb
        acc_ref[2:3] = pc
        acc_ref[3:4] = pd
        acc_ref[4:5] = num_part
        kacc_ref[0:1] = pk

    @pl.when(i > 0)
    def _accum():
        na, nb, nc, nd, nk = _comb(
            acc_ref[0:1], acc_ref[1:2], acc_ref[2:3], acc_ref[3:4],
            kacc_ref[0:1], pa, pb, pc, pd, pk)
        acc_ref[0:1] = na
        acc_ref[1:2] = nb
        acc_ref[2:3] = nc
        acc_ref[3:4] = nd
        acc_ref[4:5] = acc_ref[4:5] + num_part
        kacc_ref[0:1] = nk

    @pl.when(i == nsteps - 1)
    def _final():
        den = (jnp.log(acc_ref[0:1]) + _LN2 * kacc_ref[0:1].astype(jnp.float32)
               + sm_ref[2])
        out_ref[...] = jnp.sum(acc_ref[4:5] - den, axis=(0, 1), keepdims=True)


def kernel(log_probs, input_lens, labels, den_scores):
    n_b, t_len, nv = log_probs.shape
    n_cls = (den_scores.shape[0] - 4) // 2
    t_blk = 512
    n_seg = 8
    seglen = t_blk // n_seg
    n_chunk = t_len // t_blk

    w0 = jax.nn.log_softmax(den_scores[:n_cls + 3])
    w1 = jax.nn.log_softmax(den_scores[n_cls + 3:])
    sm = jnp.concatenate([
        jnp.stack([jnp.exp(w0[1 + n_cls]),
                   jnp.exp(w1[n_cls]),
                   w0[2 + n_cls],
                   jnp.exp(w0[0])]),
        jnp.exp(w0[1:1 + n_cls]),
        jnp.exp(w1[:n_cls]),
    ])

    packed = jnp.concatenate(
        [log_probs, labels.astype(jnp.float32)[:, :, None]], axis=2)
    lp_t = (packed
            .reshape(n_b, n_chunk, n_seg, seglen, nv + 1)
            .transpose(4, 1, 3, 2, 0)
            .reshape(nv + 1, t_len, n_b))
    lens2d = input_lens.reshape(1, n_b).astype(jnp.int32)

    res = pl.pallas_call(
        functools.partial(_body, n_b=n_b, t_blk=t_blk, n_seg=n_seg, nv=nv),
        grid=(n_chunk,),
        in_specs=[
            pl.BlockSpec((nv + 1, t_blk, n_b), lambda i: (0, i, 0)),
            pl.BlockSpec((1, n_b), lambda i: (0, 0)),
            pl.BlockSpec(memory_space=pltpu.SMEM),
        ],
        out_specs=pl.BlockSpec((1, 1), lambda i: (0, 0)),
        out_shape=jax.ShapeDtypeStruct((1, 1), jnp.float32),
        scratch_shapes=[
            pltpu.VMEM((8, n_b), jnp.float32),
            pltpu.VMEM((8, n_b), jnp.int32),
        ],
    )(lp_t, lens2d, sm)
    return res[0, 0]


# speedup vs baseline: 1.7085x; 1.7085x over previous
import functools

import jax
import jax.numpy as jnp
from jax.experimental import pallas as pl
from jax.experimental.pallas import tpu as pltpu

_LN2 = 0.6931471805599453


def _comb(a1, b1, c1, d1, k1, a2, b2, c2, d2, k2):
    na = a2 * a1 + b2 * c1
    nb = a2 * b1 + b2 * d1
    nc = c2 * a1 + d2 * c1
    nd = c2 * b1 + d2 * d1
    m = jnp.maximum(jnp.maximum(na, nb), jnp.maximum(nc, nd))
    ebits = jax.lax.shift_right_logical(
        jax.lax.bitcast_convert_type(m, jnp.int32), 23)
    scale = jax.lax.bitcast_convert_type(
        jax.lax.shift_left(254 - ebits, 23), jnp.float32)
    nk = k1 + k2 + (ebits - 127)
    return na * scale, nb * scale, nc * scale, nd * scale, nk


def _body(lp_ref, lab_ref, len_ref, sm_ref, out_ref, acc_ref, kacc_ref,
          *, n_b, t_blk, n_seg):
    i = pl.program_id(0)
    nsteps = pl.num_programs(0)
    lens = len_ref[...]
    labs = lab_ref[...]
    nv = lp_ref.shape[0]
    seglen = t_blk // n_seg

    r_iota = jax.lax.broadcasted_iota(jnp.int32, (t_blk, n_b), 0)
    tglob = i * t_blk + r_iota
    mask = tglob < lens

    s00 = jnp.zeros((t_blk, n_b), jnp.float32)
    s10 = jnp.zeros((t_blk, n_b), jnp.float32)
    emit = jnp.zeros((t_blk, n_b), jnp.float32)
    p_i = None
    for v in range(nv):
        ev = lp_ref[v]
        emit = jnp.where(labs == v, ev, emit)
        if v == 0:
            continue
        pv = jnp.exp(ev)
        if v == 1:
            s00 = s00 + sm_ref[3] * pv
        elif v == 2:
            p_i = pv
        else:
            s00 = s00 + sm_ref[4 + v - 3] * pv
            s10 = s10 + sm_ref[14 + v - 3] * pv

    num_part = jnp.sum(jnp.where(mask, emit, 0.0), axis=0, keepdims=True)

    a = jnp.where(mask, s00, 1.0)
    b = jnp.where(mask, s10, 0.0)
    c = jnp.where(mask, sm_ref[0] * p_i, 0.0)
    d = jnp.where(mask, sm_ref[1] * p_i, 1.0)

    a3 = jnp.transpose(a.reshape(n_seg, seglen, n_b), (1, 0, 2))
    b3 = jnp.transpose(b.reshape(n_seg, seglen, n_b), (1, 0, 2))
    c3 = jnp.transpose(c.reshape(n_seg, seglen, n_b), (1, 0, 2))
    d3 = jnp.transpose(d.reshape(n_seg, seglen, n_b), (1, 0, 2))

    ra, rb, rc, rd = a3[0], b3[0], c3[0], d3[0]
    rk = jnp.zeros((n_seg, n_b), jnp.int32)
    zk = rk
    for o in range(1, seglen):
        ra, rb, rc, rd, rk = _comb(ra, rb, rc, rd, rk,
                                   a3[o], b3[o], c3[o], d3[o], zk)

    pa, pb, pc, pd = ra[0:1], rb[0:1], rc[0:1], rd[0:1]
    pk = rk[0:1]
    for s in range(1, n_seg):
        pa, pb, pc, pd, pk = _comb(pa, pb, pc, pd, pk,
                                   ra[s:s + 1], rb[s:s + 1],
                                   rc[s:s + 1], rd[s:s + 1], rk[s:s + 1])

    @pl.when(i == 0)
    def _init():
        acc_ref[0:1] = pa
        acc_ref[1:2] = pb
        acc_ref[2:3] = pc
        acc_ref[3:4] = pd
        acc_ref[4:5] = num_part
        kacc_ref[0:1] = pk

    @pl.when(i > 0)
    def _accum():
        na, nb, nc, nd, nk = _comb(
            acc_ref[0:1], acc_ref[1:2], acc_ref[2:3], acc_ref[3:4],
            kacc_ref[0:1], pa, pb, pc, pd, pk)
        acc_ref[0:1] = na
        acc_ref[1:2] = nb
        acc_ref[2:3] = nc
        acc_ref[3:4] = nd
        acc_ref[4:5] = acc_ref[4:5] + num_part
        kacc_ref[0:1] = nk

    @pl.when(i == nsteps - 1)
    def _final():
        den = (jnp.log(acc_ref[0:1]) + _LN2 * kacc_ref[0:1].astype(jnp.float32)
               + sm_ref[2])
        out_ref[...] = jnp.sum(acc_ref[4:5] - den, axis=(0, 1), keepdims=True)


def kernel(log_probs, input_lens, labels, den_scores):
    n_b, t_len, nv = log_probs.shape
    n_cls = (den_scores.shape[0] - 4) // 2
    t_blk = 512
    n_seg = 8
    seglen = t_blk // n_seg
    n_chunk = t_len // t_blk

    w0 = jax.nn.log_softmax(den_scores[:n_cls + 3])
    w1 = jax.nn.log_softmax(den_scores[n_cls + 3:])
    sm = jnp.concatenate([
        jnp.stack([jnp.exp(w0[1 + n_cls]),
                   jnp.exp(w1[n_cls]),
                   w0[2 + n_cls],
                   jnp.exp(w0[0])]),
        jnp.exp(w0[1:1 + n_cls]),
        jnp.exp(w1[:n_cls]),
    ])

    lp_t = jnp.transpose(log_probs, (2, 1, 0))
    labels_t = jnp.transpose(labels.astype(jnp.int32), (1, 0))
    lens2d = input_lens.reshape(1, n_b).astype(jnp.int32)

    res = pl.pallas_call(
        functools.partial(_body, n_b=n_b, t_blk=t_blk, n_seg=n_seg),
        grid=(n_chunk,),
        in_specs=[
            pl.BlockSpec((nv, t_blk, n_b), lambda i: (0, i, 0)),
            pl.BlockSpec((t_blk, n_b), lambda i: (i, 0)),
            pl.BlockSpec((1, n_b), lambda i: (0, 0)),
            pl.BlockSpec(memory_space=pltpu.SMEM),
        ],
        out_specs=pl.BlockSpec((1, 1), lambda i: (0, 0)),
        out_shape=jax.ShapeDtypeStruct((1, 1), jnp.float32),
        scratch_shapes=[
            pltpu.VMEM((8, n_b), jnp.float32),
            pltpu.VMEM((8, n_b), jnp.int32),
        ],
    )(lp_t, labels_t, lens2d, sm)
    return res[0, 0]
